# baseline (speedup 1.0000x reference)
import functools

import jax
import jax.numpy as jnp
import numpy as np
from jax import lax
from jax.experimental import pallas as pl
from jax.experimental.pallas import tpu as pltpu
from jax.experimental.pallas import tpu_sc as plsc

B = 4096
F = 26
NUM_ROWS_PER_FIELD = 100000
NW = 32
BPW = B // NW
NPW = BPW * F
LANES = 16
NVEC = NPW // LANES
CPW = BPW // LANES

_OFFS_TILE = np.tile(np.arange(F, dtype=np.int32) * NUM_ROWS_PER_FIELD, BPW)


def _make_kernel():
    mesh = plsc.VectorSubcoreMesh(core_axis_name="c", subcore_axis_name="s")

    @functools.partial(
        pl.kernel,
        out_type=jax.ShapeDtypeStruct((B,), jnp.float32),
        mesh=mesh,
        scratch_types=[
            pltpu.VMEM((NPW,), jnp.int32),
            pltpu.VMEM((NPW,), jnp.int32),
            pltpu.VMEM((NPW,), jnp.float32),
            pltpu.VMEM((LANES,), jnp.float32),
            pltpu.VMEM((BPW,), jnp.float32),
            pltpu.SemaphoreType.DMA,
        ],
        compiler_params=pltpu.CompilerParams(needs_layout_passes=False),
    )
    def k(x_hbm, offs_hbm, table_hbm, bias_hbm, out_hbm,
          idx_v, offs_v, vals_v, bias_v, out_v, sem):
        wid = lax.axis_index("s") * 2 + lax.axis_index("c")
        base = wid * NPW

        pltpu.sync_copy(x_hbm.at[pl.ds(base, NPW)], idx_v)
        pltpu.sync_copy(offs_hbm, offs_v)
        pltpu.sync_copy(bias_hbm, bias_v)

        for i in range(NVEC):
            sl = pl.ds(i * LANES, LANES)
            idx_v[sl] = idx_v[sl] + offs_v[sl]

        pltpu.async_copy(table_hbm.at[idx_v], vals_v, sem).wait()

        lane26 = lax.iota(jnp.int32, 16) * F
        bias16 = bias_v[...]
        for c in range(CPW):
            acc = bias16
            cbase = c * LANES * F
            for f in range(F):
                acc = acc + plsc.load_gather(vals_v, [lane26 + (cbase + f)])
            out_v[pl.ds(c * LANES, LANES)] = acc

        pltpu.sync_copy(out_v, out_hbm.at[pl.ds(wid * BPW, BPW)])

    return k


_sc_kernel = _make_kernel()


def kernel(x, table, bias):
    xf = x.reshape(-1)
    tablef = table.reshape(-1)
    offs = jnp.asarray(_OFFS_TILE)
    bias16 = jnp.broadcast_to(bias.astype(jnp.float32), (LANES,))
    out = _sc_kernel(xf, offs, tablef, bias16)
    return out.reshape(B, 1)


# speedup vs baseline: 1.0288x; 1.0288x over previous
import functools

import jax
import jax.numpy as jnp
from jax import lax
from jax.experimental import pallas as pl
from jax.experimental.pallas import tpu as pltpu
from jax.experimental.pallas import tpu_sc as plsc

B = 4096
F = 26
NUM_ROWS_PER_FIELD = 100000
NW = 32
BPW = B // NW
NPW = BPW * F
LANES = 16
NCHUNK = 4
RPC = BPW // NCHUNK
NPC = RPC * F
VPF = RPC // LANES
CVEC = NPC // LANES


def _make_kernel():
    mesh = plsc.VectorSubcoreMesh(core_axis_name="c", subcore_axis_name="s")

    @functools.partial(
        pl.kernel,
        out_type=jax.ShapeDtypeStruct((B,), jnp.float32),
        mesh=mesh,
        scratch_types=[
            pltpu.VMEM((NPW,), jnp.int32),
            pltpu.VMEM((NPW,), jnp.float32),
            pltpu.VMEM((LANES,), jnp.float32),
            pltpu.VMEM((BPW,), jnp.float32),
            [pltpu.SemaphoreType.DMA] * NCHUNK,
        ],
    )
    def k(xt_hbm, table_hbm, bias_hbm, out_hbm,
          idx_v, vals_v, bias_v, out_v, sems):
        wid = lax.axis_index("s") * 2 + lax.axis_index("c")
        base = wid * NPW

        pltpu.sync_copy(xt_hbm.at[pl.ds(base, NPW)], idx_v)
        pltpu.sync_copy(bias_hbm, bias_v)

        copies = []
        for ch in range(NCHUNK):
            cb = ch * NPC
            for v in range(CVEC):
                off = jnp.int32((v // VPF) * NUM_ROWS_PER_FIELD)
                sl = pl.ds(cb + v * LANES, LANES)
                idx_v[sl] = idx_v[sl] + off
            copies.append(pltpu.async_copy(
                table_hbm.at[idx_v.at[pl.ds(cb, NPC)]],
                vals_v.at[pl.ds(cb, NPC)], sems[ch]))

        bias16 = bias_v[...]
        for ch in range(NCHUNK):
            copies[ch].wait()
            cb = ch * NPC
            for c in range(VPF):
                acc = bias16
                for f in range(F):
                    acc = acc + vals_v[pl.ds(cb + f * RPC + c * LANES, LANES)]
                out_v[pl.ds(ch * RPC + c * LANES, LANES)] = acc

        pltpu.sync_copy(out_v, out_hbm.at[pl.ds(wid * BPW, BPW)])

    return k


_sc_kernel = _make_kernel()


def kernel(x, table, bias):
    xt = x.reshape(NW, NCHUNK, RPC, F).transpose(0, 1, 3, 2).reshape(-1)
    tablef = table.reshape(-1)
    bias16 = jnp.broadcast_to(bias.astype(jnp.float32), (LANES,))
    out = _sc_kernel(xt, tablef, bias16)
    return out.reshape(B, 1)


# speedup vs baseline: 1.0318x; 1.0030x over previous
import functools

import jax
import jax.numpy as jnp
from jax import lax
from jax.experimental import pallas as pl
from jax.experimental.pallas import tpu as pltpu
from jax.experimental.pallas import tpu_sc as plsc

B = 4096
F = 26
NUM_ROWS_PER_FIELD = 100000
NW = 32
BPW = B // NW
NPW = BPW * F
LANES = 16
NVEC = NPW // LANES
VPF = BPW // LANES
CPW = BPW // LANES


def _make_kernel():
    mesh = plsc.VectorSubcoreMesh(core_axis_name="c", subcore_axis_name="s")

    @functools.partial(
        pl.kernel,
        out_type=jax.ShapeDtypeStruct((B,), jnp.float32),
        mesh=mesh,
        scratch_types=[
            pltpu.VMEM((NPW,), jnp.int32),
            pltpu.VMEM((NPW,), jnp.float32),
            pltpu.VMEM((LANES,), jnp.float32),
            pltpu.VMEM((BPW,), jnp.float32),
            pltpu.SemaphoreType.DMA,
        ],
    )
    def k(xt_hbm, table_hbm, bias_hbm, out_hbm,
          idx_v, vals_v, bias_v, out_v, sem):
        wid = lax.axis_index("s") * 2 + lax.axis_index("c")
        base = wid * NPW

        pltpu.sync_copy(xt_hbm.at[pl.ds(base, NPW)], idx_v)
        pltpu.sync_copy(bias_hbm, bias_v)

        def ibody(v, _):
            sl = pl.ds(v * LANES, LANES)
            off = (v // VPF) * NUM_ROWS_PER_FIELD
            idx_v[sl] = idx_v[sl] + off
            return 0
        lax.fori_loop(0, NVEC, ibody, 0)

        pltpu.async_copy(table_hbm.at[idx_v], vals_v, sem).wait()

        bias16 = bias_v[...]

        def cbody(c, _):
            def fbody(f, acc):
                return acc + vals_v[pl.ds(f * BPW + c * LANES, LANES)]
            acc = lax.fori_loop(0, F, fbody, bias16)
            out_v[pl.ds(c * LANES, LANES)] = acc
            return 0
        lax.fori_loop(0, CPW, cbody, 0)

        pltpu.sync_copy(out_v, out_hbm.at[pl.ds(wid * BPW, BPW)])

    return k


_sc_kernel = _make_kernel()


def kernel(x, table, bias):
    xt = x.reshape(NW, BPW, F).transpose(0, 2, 1).reshape(-1)
    tablef = table.reshape(-1)
    bias16 = jnp.broadcast_to(bias.astype(jnp.float32), (LANES,))
    out = _sc_kernel(xt, tablef, bias16)
    return out.reshape(B, 1)


# speedup vs baseline: 1.0381x; 1.0061x over previous
import functools

import jax
import jax.numpy as jnp
from jax import lax
from jax.experimental import pallas as pl
from jax.experimental.pallas import tpu as pltpu
from jax.experimental.pallas import tpu_sc as plsc

B = 4096
F = 26
NUM_ROWS_PER_FIELD = 100000
NW = 32
BPW = B // NW
NPW = BPW * F
LANES = 16
CPW = BPW // LANES


def _make_kernel():
    mesh = plsc.VectorSubcoreMesh(core_axis_name="c", subcore_axis_name="s")

    @functools.partial(
        pl.kernel,
        out_type=jax.ShapeDtypeStruct((B,), jnp.float32),
        mesh=mesh,
        scratch_types=[
            pltpu.VMEM((NPW,), jnp.int32),
            pltpu.VMEM((NPW,), jnp.float32),
            pltpu.VMEM((LANES,), jnp.float32),
            pltpu.VMEM((BPW,), jnp.float32),
            pltpu.SemaphoreType.DMA,
            pltpu.SemaphoreType.DMA,
        ],
    )
    def k(xt_hbm, table_hbm, bias_hbm, out_hbm,
          idx_v, vals_v, bias_v, out_v, sem, bsem):
        wid = lax.axis_index("s") * 2 + lax.axis_index("c")
        base = wid * NPW

        bias_cp = pltpu.async_copy(bias_hbm, bias_v, bsem)
        pltpu.sync_copy(xt_hbm.at[pl.ds(base, NPW)], idx_v)

        for f in range(F):
            off = jnp.int32(f * NUM_ROWS_PER_FIELD)
            for c in range(CPW):
                sl = pl.ds(f * BPW + c * LANES, LANES)
                idx_v[sl] = idx_v[sl] + off

        pltpu.async_copy(table_hbm.at[idx_v], vals_v, sem).wait()
        bias_cp.wait()

        bias16 = bias_v[...]
        accs = [bias16] * CPW
        for f in range(F):
            for c in range(CPW):
                accs[c] = accs[c] + vals_v[pl.ds(f * BPW + c * LANES, LANES)]
        for c in range(CPW):
            out_v[pl.ds(c * LANES, LANES)] = accs[c]

        pltpu.sync_copy(out_v, out_hbm.at[pl.ds(wid * BPW, BPW)])

    return k


_sc_kernel = _make_kernel()


def kernel(x, table, bias):
    xt = x.reshape(NW, BPW, F).transpose(0, 2, 1).reshape(-1)
    tablef = table.reshape(-1)
    bias16 = jnp.broadcast_to(bias.astype(jnp.float32), (LANES,))
    out = _sc_kernel(xt, tablef, bias16)
    return out.reshape(B, 1)


# speedup vs baseline: 1.0384x; 1.0002x over previous
import functools

import jax
import jax.numpy as jnp
from jax import lax
from jax.experimental import pallas as pl
from jax.experimental.pallas import tpu as pltpu
from jax.experimental.pallas import tpu_sc as plsc

B = 4096
F = 26
NUM_ROWS_PER_FIELD = 100000
NW = 32
BPW = B // NW
NPW = BPW * F
LANES = 16
CPW = BPW // LANES


def _make_kernel():
    mesh = plsc.VectorSubcoreMesh(core_axis_name="c", subcore_axis_name="s")

    @functools.partial(
        pl.kernel,
        out_type=jax.ShapeDtypeStruct((B,), jnp.float32),
        mesh=mesh,
        scratch_types=[
            pltpu.VMEM((NPW,), jnp.int32),
            pltpu.VMEM((NPW,), jnp.float32),
            pltpu.VMEM((LANES,), jnp.float32),
            pltpu.VMEM((BPW,), jnp.float32),
            pltpu.SemaphoreType.DMA,
            pltpu.SemaphoreType.DMA,
        ],
    )
    def k(xt_hbm, table_hbm, bias_hbm, out_hbm,
          idx_v, vals_v, bias_v, out_v, sem, bsem):
        wid = lax.axis_index("s") * 2 + lax.axis_index("c")
        base = wid * NPW

        with jax.named_scope("ph_xload"):
            bias_cp = pltpu.async_copy(bias_hbm, bias_v, bsem)
            pltpu.sync_copy(xt_hbm.at[pl.ds(base, NPW)], idx_v)

        with jax.named_scope("ph_idx"):
            for f in range(F):
                off = jnp.int32(f * NUM_ROWS_PER_FIELD)
                for c in range(CPW):
                    sl = pl.ds(f * BPW + c * LANES, LANES)
                    idx_v[sl] = idx_v[sl] + off

        with jax.named_scope("ph_gather"):
            pltpu.async_copy(table_hbm.at[idx_v], vals_v, sem).wait()
            bias_cp.wait()

        with jax.named_scope("ph_reduce"):
            bias16 = bias_v[...]
            accs = [bias16] * CPW
            for f in range(F):
                for c in range(CPW):
                    accs[c] = accs[c] + vals_v[pl.ds(f * BPW + c * LANES, LANES)]
            for c in range(CPW):
                out_v[pl.ds(c * LANES, LANES)] = accs[c]

        with jax.named_scope("ph_out"):
            pltpu.sync_copy(out_v, out_hbm.at[pl.ds(wid * BPW, BPW)])

    return k


_sc_kernel = _make_kernel()


def kernel(x, table, bias):
    xt = x.reshape(NW, BPW, F).transpose(0, 2, 1).reshape(-1)
    tablef = table.reshape(-1)
    bias16 = jnp.broadcast_to(bias.astype(jnp.float32), (LANES,))
    out = _sc_kernel(xt, tablef, bias16)
    return out.reshape(B, 1)
